# baseline (speedup 1.0000x reference)
import functools

import jax
import jax.numpy as jnp
import numpy as np
from jax import lax
from jax.experimental import pallas as pl
from jax.experimental.pallas import tpu as pltpu
from jax.experimental.pallas import tpu_sc as plsc

N = 10000
E = 320000
D_MODEL = 128
H = 8
DEPTH = D_MODEL // H
D_EDGE = 16
D_FF = 512

NC = 2
NS = 16
NW = NC * NS
EDGES_PER_W = E // NW
CH = 80
NCHUNK = EDGES_PER_W // CH
N_PAD = 10240
ROWS_PER_TILE = N_PAD // NS
RCH = 128

BN = 1000
BE = 3200

def _dot(a, b):
    return jnp.dot(a, b, preferred_element_type=jnp.float32)


def _dot_f32(a, b):
    ah = a.astype(jnp.bfloat16).astype(jnp.float32)
    al = a - ah
    return (jnp.dot(ah, b, preferred_element_type=jnp.float32)
            + jnp.dot(al, b, preferred_element_type=jnp.float32))


def _lane_sum(v):
    s = v[:, 0:8]
    for i in range(1, 16):
        s = s + v[:, 8 * i:8 * i + 8]
    s = s[:, 0:4] + s[:, 4:8]
    s = s[:, 0:2] + s[:, 2:4]
    return s[:, 0:1] + s[:, 1:2]


def _ln_match(x, g, b):
    m = _lane_sum(x) * jnp.float32(1.0 / D_MODEL)
    c = x - m
    var = _lane_sum(c * c) * jnp.float32(1.0 / D_MODEL)
    return c / jnp.sqrt(var + 1e-6) * g + b


def _p1_body(feat, g1, b1, Wq, bq, Wk, bk, Wv, bv, q_o, k_o, v_o, ksum_o):
    i = pl.program_id(0)
    x = feat[...]
    xn = _ln_match(x, g1[...], b1[...])
    qb = _dot(xn, Wq[...]) + bq[...]
    kb = _dot(xn, Wk[...]) + bk[...]
    vb = _dot(xn, Wv[...]) + bv[...]
    q_o[...] = qb
    k_o[...] = kb
    v_o[...] = vb
    part = jnp.sum(kb, axis=0, keepdims=True)

    @pl.when(i == 0)
    def _():
        ksum_o[...] = part

    @pl.when(i > 0)
    def _():
        ksum_o[...] = ksum_o[...] + part


def _p1(feature, g1, b1, Wq, bq, Wk, bk, Wv, bv):
    grid = N // BN
    full = lambda shape: pl.BlockSpec(shape, lambda i: (0, 0))
    return pl.pallas_call(
        _p1_body,
        grid=(grid,),
        in_specs=[
            pl.BlockSpec((BN, D_MODEL), lambda i: (i, 0)),
            full((1, D_MODEL)), full((1, D_MODEL)),
            full((D_MODEL, D_MODEL)), full((1, D_MODEL)),
            full((D_MODEL, D_MODEL)), full((1, D_MODEL)),
            full((D_MODEL, D_MODEL)), full((1, D_MODEL)),
        ],
        out_specs=[
            pl.BlockSpec((BN, D_MODEL), lambda i: (i, 0)),
            pl.BlockSpec((BN, D_MODEL), lambda i: (i, 0)),
            pl.BlockSpec((BN, D_MODEL), lambda i: (i, 0)),
            pl.BlockSpec((1, D_MODEL), lambda i: (0, 0)),
        ],
        out_shape=[
            jax.ShapeDtypeStruct((N, D_MODEL), jnp.float32),
            jax.ShapeDtypeStruct((N, D_MODEL), jnp.float32),
            jax.ShapeDtypeStruct((N, D_MODEL), jnp.float32),
            jax.ShapeDtypeStruct((1, D_MODEL), jnp.float32),
        ],
    )(feature, g1, b1, Wq, bq, Wk, bk, Wv, bv)


def _p2_body(q_hbm, k_hbm, v_hbm, row_hbm, col_hbm, zero_hbm,
             qg_hbm, kg_hbm, part_hbm,
             colv, rowv, qbuf, kbuf, vbuf, zbuf, acc, sem):
    c = lax.axis_index("c")
    s = lax.axis_index("s")

    pltpu.sync_copy(zero_hbm, zbuf)
    base_r = s * ROWS_PER_TILE
    for t in range(ROWS_PER_TILE // RCH):
        pltpu.sync_copy(zbuf, acc.at[pl.ds(base_r + t * RCH, RCH)])
    plsc.subcore_barrier()

    wbase = (c * NS + s) * EDGES_PER_W

    def step(j, carry):
        base = wbase + j * CH
        pltpu.sync_copy(col_hbm.at[pl.ds(base, CH)], colv)
        pltpu.sync_copy(row_hbm.at[pl.ds(base, CH)], rowv)
        pltpu.async_copy(q_hbm.at[colv], qbuf, sem).wait()
        pltpu.sync_copy(qbuf, qg_hbm.at[pl.ds(base, CH)])
        pltpu.async_copy(k_hbm.at[rowv], kbuf, sem).wait()
        pltpu.sync_copy(kbuf, kg_hbm.at[pl.ds(base, CH)])
        pltpu.async_copy(v_hbm.at[rowv], vbuf, sem).wait()
        pltpu.sync_copy(vbuf, acc.at[colv], add=True)
        return carry

    lax.fori_loop(0, NCHUNK, step, 0)
    plsc.subcore_barrier()

    for t in range(ROWS_PER_TILE // RCH):
        r0 = base_r + t * RCH
        pltpu.sync_copy(acc.at[pl.ds(r0, RCH)], zbuf)
        pltpu.sync_copy(zbuf, part_hbm.at[pl.ds(c * N_PAD + r0, RCH)])


def _p2(q, k, v, row, col, zeros):
    mesh = plsc.VectorSubcoreMesh(core_axis_name="c", subcore_axis_name="s")
    fn = pl.kernel(
        _p2_body,
        out_type=(
            jax.ShapeDtypeStruct((E, D_MODEL), jnp.float32),
            jax.ShapeDtypeStruct((E, D_MODEL), jnp.float32),
            jax.ShapeDtypeStruct((NC * N_PAD, D_MODEL), jnp.float32),
        ),
        mesh=mesh,
        scratch_types=[
            pltpu.VMEM((CH,), jnp.int32),
            pltpu.VMEM((CH,), jnp.int32),
            pltpu.VMEM((CH, D_MODEL), jnp.float32),
            pltpu.VMEM((CH, D_MODEL), jnp.float32),
            pltpu.VMEM((CH, D_MODEL), jnp.float32),
            pltpu.VMEM((RCH, D_MODEL), jnp.float32),
            pltpu.VMEM_SHARED((N_PAD, D_MODEL), jnp.float32),
            pltpu.SemaphoreType.DMA,
        ],
    )
    return fn(q, k, v, row, col, zeros)


def _p3_body(qg, kg, er, sp, ksum, Wrel, brel, wsp, bsp, attw_o):
    rel = _dot(er[...], Wrel[...]) + brel[...]
    qe = qg[...] + rel
    ke = kg[...] + rel
    rows = lax.broadcasted_iota(jnp.int32, (D_MODEL, H), 0) // DEPTH
    cols = lax.broadcasted_iota(jnp.int32, (D_MODEL, H), 1)
    hm = (rows == cols).astype(jnp.float32)
    c = jnp.float32(1.0 / np.sqrt(np.sqrt(float(H))))
    num = _dot_f32(qe * ke, hm) * c + sp[...] * wsp[...] + bsp[...]
    qg16 = qg[...].astype(jnp.bfloat16).astype(jnp.float32)
    ks16 = ksum[...].astype(jnp.bfloat16).astype(jnp.float32)
    norm = _dot_f32(qg16 * ks16, hm)
    attw_o[...] = num / norm


def _p3(qg, kg, edge_rel, sp_value, ksum, Wrel, brel, wsp, bsp):
    grid = E // BE
    full = lambda shape: pl.BlockSpec(shape, lambda i: (0, 0))
    return pl.pallas_call(
        _p3_body,
        grid=(grid,),
        in_specs=[
            pl.BlockSpec((BE, D_MODEL), lambda i: (i, 0)),
            pl.BlockSpec((BE, D_MODEL), lambda i: (i, 0)),
            pl.BlockSpec((BE, D_EDGE), lambda i: (i, 0)),
            pl.BlockSpec((BE, 1), lambda i: (i, 0)),
            full((1, D_MODEL)),
            full((D_EDGE, D_MODEL)), full((1, D_MODEL)),
            full((1, H)), full((1, H)),
        ],
        out_specs=pl.BlockSpec((BE, H), lambda i: (i, 0)),
        out_shape=jax.ShapeDtypeStruct((E, H), jnp.float32),
    )(qg, kg, edge_rel, sp_value, ksum, Wrel, brel, wsp, bsp)


def _p4_body(part, feat, Wd, bd, g2, b2, W1, bf1, W2, bf2, out_o):
    agg = part[0] + part[1]
    attn_out = _dot(agg, Wd[...]) + bd[...]
    out1 = attn_out + feat[...]
    m = jnp.mean(out1, axis=-1, keepdims=True)
    var = jnp.mean(jnp.square(out1 - m), axis=-1, keepdims=True)
    t = (out1 - m) / jnp.sqrt(var + 1e-6) * g2[...] + b2[...]
    ffn = _dot(jnp.maximum(_dot(t, W1[...]) + bf1[...], 0.0), W2[...]) + bf2[...]
    out_o[...] = out1 + ffn


def _p4(partials, feature, Wd, bd, g2, b2, W1, bf1, W2, bf2):
    grid = N // BN
    full = lambda shape: pl.BlockSpec(shape, lambda *_: tuple(0 for _ in shape))
    return pl.pallas_call(
        _p4_body,
        grid=(grid,),
        in_specs=[
            pl.BlockSpec((2, BN, D_MODEL), lambda i: (0, i, 0)),
            pl.BlockSpec((BN, D_MODEL), lambda i: (i, 0)),
            full((D_MODEL, D_MODEL)), full((1, D_MODEL)),
            full((1, D_MODEL)), full((1, D_MODEL)),
            full((D_MODEL, D_FF)), full((1, D_FF)),
            full((D_FF, D_MODEL)), full((1, D_MODEL)),
        ],
        out_specs=pl.BlockSpec((BN, D_MODEL), lambda i: (i, 0)),
        out_shape=jax.ShapeDtypeStruct((N, D_MODEL), jnp.float32),
    )(partials, feature, Wd, bd, g2, b2, W1, bf1, W2, bf2)


def kernel(feature, sp_edge_index, sp_value, edge_rel, g1, b1, g2, b2,
           Wq, bq, Wk, bk, Wv, bv, Wd, bd, Wrel, brel, Wsp, bsp,
           W1, bf1, W2, bf2):
    r2 = lambda a: a.reshape(1, -1)
    q, k, v, ksum = _p1(feature, r2(g1), r2(b1), Wq, r2(bq), Wk, r2(bk),
                        Wv, r2(bv))
    row = sp_edge_index[0]
    col = sp_edge_index[1]
    zeros = jnp.zeros((RCH, D_MODEL), jnp.float32)
    qg, kg, partials = _p2(q, k, v, row, col, zeros)
    attw = _p3(qg, kg, edge_rel, sp_value, ksum, Wrel, r2(brel),
               Wsp.reshape(1, H), r2(bsp))
    partials = partials.reshape(NC, N_PAD, D_MODEL)[:, :N, :]
    out2 = _p4(partials, feature, Wd, r2(bd), r2(g2), r2(b2),
               W1, r2(bf1), W2, r2(bf2))
    return (out2, attw)


# speedup vs baseline: 1.2745x; 1.2745x over previous
import functools

import jax
import jax.numpy as jnp
import numpy as np
from jax import lax
from jax.experimental import pallas as pl
from jax.experimental.pallas import tpu as pltpu
from jax.experimental.pallas import tpu_sc as plsc

N = 10000
E = 320000
D_MODEL = 128
H = 8
DEPTH = D_MODEL // H
D_EDGE = 16
D_FF = 512

NC = 2
NS = 16
NW = NC * NS
EDGES_PER_W = E // NW
CH = 40
NCHUNK = EDGES_PER_W // CH
N_PAD = 10240
ROWS_PER_TILE = N_PAD // NS
RCH = CH

BN = 1000
BE = 3200

def _dot(a, b):
    return jnp.dot(a, b, preferred_element_type=jnp.float32)


def _dot_f32(a, b):
    ah = a.astype(jnp.bfloat16).astype(jnp.float32)
    al = a - ah
    return (jnp.dot(ah, b, preferred_element_type=jnp.float32)
            + jnp.dot(al, b, preferred_element_type=jnp.float32))


def _lane_sum(v):
    s = v[:, 0:8]
    for i in range(1, 16):
        s = s + v[:, 8 * i:8 * i + 8]
    s = s[:, 0:4] + s[:, 4:8]
    s = s[:, 0:2] + s[:, 2:4]
    return s[:, 0:1] + s[:, 1:2]


def _ln_match(x, g, b):
    m = _lane_sum(x) * jnp.float32(1.0 / D_MODEL)
    c = x - m
    var = _lane_sum(c * c) * jnp.float32(1.0 / D_MODEL)
    return c / jnp.sqrt(var + 1e-6) * g + b


def _p1_body(feat, g1, b1, Wq, bq, Wk, bk, Wv, bv, q_o, k_o, v_o, ksum_o):
    i = pl.program_id(0)
    x = feat[...]
    xn = _ln_match(x, g1[...], b1[...])
    qb = _dot(xn, Wq[...]) + bq[...]
    kb = _dot(xn, Wk[...]) + bk[...]
    vb = _dot(xn, Wv[...]) + bv[...]
    q_o[...] = qb
    k_o[...] = kb
    v_o[...] = vb
    part = jnp.sum(kb, axis=0, keepdims=True)

    @pl.when(i == 0)
    def _():
        ksum_o[...] = part

    @pl.when(i > 0)
    def _():
        ksum_o[...] = ksum_o[...] + part


def _p1(feature, g1, b1, Wq, bq, Wk, bk, Wv, bv):
    grid = N // BN
    full = lambda shape: pl.BlockSpec(shape, lambda i: (0, 0))
    return pl.pallas_call(
        _p1_body,
        grid=(grid,),
        in_specs=[
            pl.BlockSpec((BN, D_MODEL), lambda i: (i, 0)),
            full((1, D_MODEL)), full((1, D_MODEL)),
            full((D_MODEL, D_MODEL)), full((1, D_MODEL)),
            full((D_MODEL, D_MODEL)), full((1, D_MODEL)),
            full((D_MODEL, D_MODEL)), full((1, D_MODEL)),
        ],
        out_specs=[
            pl.BlockSpec((BN, D_MODEL), lambda i: (i, 0)),
            pl.BlockSpec((BN, D_MODEL), lambda i: (i, 0)),
            pl.BlockSpec((BN, D_MODEL), lambda i: (i, 0)),
            pl.BlockSpec((1, D_MODEL), lambda i: (0, 0)),
        ],
        out_shape=[
            jax.ShapeDtypeStruct((N, D_MODEL), jnp.float32),
            jax.ShapeDtypeStruct((N, D_MODEL), jnp.float32),
            jax.ShapeDtypeStruct((N, D_MODEL), jnp.float32),
            jax.ShapeDtypeStruct((1, D_MODEL), jnp.float32),
        ],
    )(feature, g1, b1, Wq, bq, Wk, bk, Wv, bv)


def _p2_body(q_hbm, k_hbm, v_hbm, row_hbm, col_hbm, zero_hbm,
             qg_hbm, kg_hbm, part_hbm,
             colv, rowv, qb0, kb0, vb0, qb1, kb1, vb1, acc,
             sem0, sem1):
    c = lax.axis_index("c")
    s = lax.axis_index("s")
    wid = c * NS + s

    pltpu.sync_copy(zero_hbm, qb0)
    base_r = s * ROWS_PER_TILE
    for t in range(ROWS_PER_TILE // RCH):
        pltpu.sync_copy(qb0, acc.at[pl.ds(base_r + t * RCH, RCH)])
    plsc.subcore_barrier()

    wbase = wid * EDGES_PER_W
    bufs = ((qb0, kb0, vb0, sem0), (qb1, kb1, vb1, sem1))

    def issue(j, p):
        qb, kb, vb, sem = bufs[p]
        base = wbase + j * CH
        pltpu.sync_copy(col_hbm.at[pl.ds(base, CH)], colv.at[p])
        pltpu.sync_copy(row_hbm.at[pl.ds(base, CH)], rowv.at[p])
        pltpu.async_copy(q_hbm.at[colv.at[p]], qb, sem)
        pltpu.async_copy(k_hbm.at[rowv.at[p]], kb, sem)
        pltpu.async_copy(v_hbm.at[rowv.at[p]], vb, sem)

    def drain(j, p):
        qb, kb, vb, sem = bufs[p]
        base = wbase + j * CH
        pltpu.make_async_copy(q_hbm.at[colv.at[p]], qb, sem).wait()
        pltpu.make_async_copy(k_hbm.at[rowv.at[p]], kb, sem).wait()
        pltpu.make_async_copy(v_hbm.at[rowv.at[p]], vb, sem).wait()
        pltpu.sync_copy(qb, qg_hbm.at[pl.ds(base, CH)])
        pltpu.sync_copy(kb, kg_hbm.at[pl.ds(base, CH)])
        pltpu.sync_copy(vb, acc.at[colv.at[p]], add=True)

    assert NCHUNK % 2 == 0
    issue(0, 0)

    def step(i, carry):
        j = 2 * i
        issue(j + 1, 1)
        drain(j, 0)
        issue(j + 2, 0)
        drain(j + 1, 1)
        return carry

    lax.fori_loop(0, (NCHUNK - 2) // 2, step, 0)
    issue(NCHUNK - 1, 1)
    drain(NCHUNK - 2, 0)
    drain(NCHUNK - 1, 1)
    plsc.subcore_barrier()

    for t in range(ROWS_PER_TILE // RCH):
        r0 = base_r + t * RCH
        pltpu.sync_copy(acc.at[pl.ds(r0, RCH)], qb0)
        pltpu.sync_copy(qb0, part_hbm.at[pl.ds(c * N_PAD + r0, RCH)])


def _p2(q, k, v, row, col, zeros):
    mesh = plsc.VectorSubcoreMesh(core_axis_name="c", subcore_axis_name="s")
    fn = pl.kernel(
        _p2_body,
        out_type=(
            jax.ShapeDtypeStruct((E, D_MODEL), jnp.float32),
            jax.ShapeDtypeStruct((E, D_MODEL), jnp.float32),
            jax.ShapeDtypeStruct((NC * N_PAD, D_MODEL), jnp.float32),
        ),
        mesh=mesh,
        scratch_types=[
            pltpu.VMEM((2, CH), jnp.int32),
            pltpu.VMEM((2, CH), jnp.int32),
            pltpu.VMEM((CH, D_MODEL), jnp.float32),
            pltpu.VMEM((CH, D_MODEL), jnp.float32),
            pltpu.VMEM((CH, D_MODEL), jnp.float32),
            pltpu.VMEM((CH, D_MODEL), jnp.float32),
            pltpu.VMEM((CH, D_MODEL), jnp.float32),
            pltpu.VMEM((CH, D_MODEL), jnp.float32),
            pltpu.VMEM_SHARED((N_PAD, D_MODEL), jnp.float32),
            pltpu.SemaphoreType.DMA,
            pltpu.SemaphoreType.DMA,
        ],
    )
    return fn(q, k, v, row, col, zeros)


def _p3_body(qg, kg, er, sp, ksum, Wrel, brel, wsp, bsp, attw_o):
    rel = _dot(er[...], Wrel[...]) + brel[...]
    qe = qg[...] + rel
    ke = kg[...] + rel
    rows = lax.broadcasted_iota(jnp.int32, (D_MODEL, H), 0) // DEPTH
    cols = lax.broadcasted_iota(jnp.int32, (D_MODEL, H), 1)
    hm = (rows == cols).astype(jnp.float32)
    c = jnp.float32(1.0 / np.sqrt(np.sqrt(float(H))))
    num = _dot_f32(qe * ke, hm) * c + sp[...] * wsp[...] + bsp[...]
    qg16 = qg[...].astype(jnp.bfloat16).astype(jnp.float32)
    ks16 = ksum[...].astype(jnp.bfloat16).astype(jnp.float32)
    norm = _dot_f32(qg16 * ks16, hm)
    attw_o[...] = num / norm


def _p3(qg, kg, edge_rel, sp_value, ksum, Wrel, brel, wsp, bsp):
    grid = E // BE
    full = lambda shape: pl.BlockSpec(shape, lambda i: (0, 0))
    return pl.pallas_call(
        _p3_body,
        grid=(grid,),
        in_specs=[
            pl.BlockSpec((BE, D_MODEL), lambda i: (i, 0)),
            pl.BlockSpec((BE, D_MODEL), lambda i: (i, 0)),
            pl.BlockSpec((BE, D_EDGE), lambda i: (i, 0)),
            pl.BlockSpec((BE, 1), lambda i: (i, 0)),
            full((1, D_MODEL)),
            full((D_EDGE, D_MODEL)), full((1, D_MODEL)),
            full((1, H)), full((1, H)),
        ],
        out_specs=pl.BlockSpec((BE, H), lambda i: (i, 0)),
        out_shape=jax.ShapeDtypeStruct((E, H), jnp.float32),
    )(qg, kg, edge_rel, sp_value, ksum, Wrel, brel, wsp, bsp)


def _p4_body(part, feat, Wd, bd, g2, b2, W1, bf1, W2, bf2, out_o):
    agg = part[0] + part[1]
    attn_out = _dot(agg, Wd[...]) + bd[...]
    out1 = attn_out + feat[...]
    m = jnp.mean(out1, axis=-1, keepdims=True)
    var = jnp.mean(jnp.square(out1 - m), axis=-1, keepdims=True)
    t = (out1 - m) / jnp.sqrt(var + 1e-6) * g2[...] + b2[...]
    ffn = _dot(jnp.maximum(_dot(t, W1[...]) + bf1[...], 0.0), W2[...]) + bf2[...]
    out_o[...] = out1 + ffn


def _p4(partials, feature, Wd, bd, g2, b2, W1, bf1, W2, bf2):
    grid = N // BN
    full = lambda shape: pl.BlockSpec(shape, lambda *_: tuple(0 for _ in shape))
    return pl.pallas_call(
        _p4_body,
        grid=(grid,),
        in_specs=[
            pl.BlockSpec((2, BN, D_MODEL), lambda i: (0, i, 0)),
            pl.BlockSpec((BN, D_MODEL), lambda i: (i, 0)),
            full((D_MODEL, D_MODEL)), full((1, D_MODEL)),
            full((1, D_MODEL)), full((1, D_MODEL)),
            full((D_MODEL, D_FF)), full((1, D_FF)),
            full((D_FF, D_MODEL)), full((1, D_MODEL)),
        ],
        out_specs=pl.BlockSpec((BN, D_MODEL), lambda i: (i, 0)),
        out_shape=jax.ShapeDtypeStruct((N, D_MODEL), jnp.float32),
    )(partials, feature, Wd, bd, g2, b2, W1, bf1, W2, bf2)


def kernel(feature, sp_edge_index, sp_value, edge_rel, g1, b1, g2, b2,
           Wq, bq, Wk, bk, Wv, bv, Wd, bd, Wrel, brel, Wsp, bsp,
           W1, bf1, W2, bf2):
    r2 = lambda a: a.reshape(1, -1)
    q, k, v, ksum = _p1(feature, r2(g1), r2(b1), Wq, r2(bq), Wk, r2(bk),
                        Wv, r2(bv))
    row = sp_edge_index[0]
    col = sp_edge_index[1]
    zeros = jnp.zeros((RCH, D_MODEL), jnp.float32)
    qg, kg, partials = _p2(q, k, v, row, col, zeros)
    attw = _p3(qg, kg, edge_rel, sp_value, ksum, Wrel, r2(brel),
               Wsp.reshape(1, H), r2(bsp))
    partials = partials.reshape(NC, N_PAD, D_MODEL)[:, :N, :]
    out2 = _p4(partials, feature, Wd, r2(bd), r2(g2), r2(b2),
               W1, r2(bf1), W2, r2(bf2))
    return (out2, attw)


# speedup vs baseline: 1.2776x; 1.0024x over previous
import functools

import jax
import jax.numpy as jnp
import numpy as np
from jax import lax
from jax.experimental import pallas as pl
from jax.experimental.pallas import tpu as pltpu
from jax.experimental.pallas import tpu_sc as plsc

N = 10000
E = 320000
D_MODEL = 128
H = 8
DEPTH = D_MODEL // H
D_EDGE = 16
D_FF = 512

NC = 2
NS = 16
NW = NC * NS
EDGES_PER_W = E // NW
CH = 40
NCHUNK = EDGES_PER_W // CH
N_PAD = 10240
ROWS_PER_TILE = N_PAD // NS
RCH = CH

BN = 1000
BE = 3200

def _dot(a, b):
    return jnp.dot(a, b, preferred_element_type=jnp.float32)


def _dot_f32(a, b):
    ah = a.astype(jnp.bfloat16).astype(jnp.float32)
    al = a - ah
    return jnp.dot(jnp.concatenate((ah, al), axis=1),
                   jnp.concatenate((b, b), axis=0),
                   preferred_element_type=jnp.float32)


def _lane_sum(v):
    s = v[:, 0:8]
    for i in range(1, 16):
        s = s + v[:, 8 * i:8 * i + 8]
    s = s[:, 0:4] + s[:, 4:8]
    s = s[:, 0:2] + s[:, 2:4]
    return s[:, 0:1] + s[:, 1:2]


def _ln_match(x, g, b):
    m = _lane_sum(x) * jnp.float32(1.0 / D_MODEL)
    c = x - m
    var = _lane_sum(c * c) * jnp.float32(1.0 / D_MODEL)
    return c / jnp.sqrt(var + 1e-6) * g + b


def _p1_body(feat, g1, b1, Wq, bq, Wk, bk, Wv, bv, q_o, k_o, v_o, ksum_o):
    i = pl.program_id(0)
    x = feat[...]
    xn = _ln_match(x, g1[...], b1[...])
    qb = _dot(xn, Wq[...]) + bq[...]
    kb = _dot(xn, Wk[...]) + bk[...]
    vb = _dot(xn, Wv[...]) + bv[...]
    q_o[...] = qb
    k_o[...] = kb
    v_o[...] = vb
    part = jnp.sum(kb, axis=0, keepdims=True)

    @pl.when(i == 0)
    def _():
        ksum_o[...] = part

    @pl.when(i > 0)
    def _():
        ksum_o[...] = ksum_o[...] + part


def _p1(feature, g1, b1, Wq, bq, Wk, bk, Wv, bv):
    grid = N // BN
    full = lambda shape: pl.BlockSpec(shape, lambda i: (0, 0))
    return pl.pallas_call(
        _p1_body,
        grid=(grid,),
        in_specs=[
            pl.BlockSpec((BN, D_MODEL), lambda i: (i, 0)),
            full((1, D_MODEL)), full((1, D_MODEL)),
            full((D_MODEL, D_MODEL)), full((1, D_MODEL)),
            full((D_MODEL, D_MODEL)), full((1, D_MODEL)),
            full((D_MODEL, D_MODEL)), full((1, D_MODEL)),
        ],
        out_specs=[
            pl.BlockSpec((BN, D_MODEL), lambda i: (i, 0)),
            pl.BlockSpec((BN, D_MODEL), lambda i: (i, 0)),
            pl.BlockSpec((BN, D_MODEL), lambda i: (i, 0)),
            pl.BlockSpec((1, D_MODEL), lambda i: (0, 0)),
        ],
        out_shape=[
            jax.ShapeDtypeStruct((N, D_MODEL), jnp.float32),
            jax.ShapeDtypeStruct((N, D_MODEL), jnp.float32),
            jax.ShapeDtypeStruct((N, D_MODEL), jnp.float32),
            jax.ShapeDtypeStruct((1, D_MODEL), jnp.float32),
        ],
    )(feature, g1, b1, Wq, bq, Wk, bk, Wv, bv)


def _p2_body(q_hbm, k_hbm, v_hbm, row_hbm, col_hbm, zero_hbm,
             qg_hbm, kg_hbm, part_hbm,
             colv, rowv, qb0, kb0, vb0, qb1, kb1, vb1, acc,
             sem0, sem1):
    c = lax.axis_index("c")
    s = lax.axis_index("s")
    wid = c * NS + s

    pltpu.sync_copy(zero_hbm, qb0)
    base_r = s * ROWS_PER_TILE
    for t in range(ROWS_PER_TILE // RCH):
        pltpu.sync_copy(qb0, acc.at[pl.ds(base_r + t * RCH, RCH)])
    plsc.subcore_barrier()

    wbase = wid * EDGES_PER_W
    bufs = ((qb0, kb0, vb0, sem0), (qb1, kb1, vb1, sem1))

    def issue(j, p):
        qb, kb, vb, sem = bufs[p]
        base = wbase + j * CH
        pltpu.sync_copy(col_hbm.at[pl.ds(base, CH)], colv.at[p])
        pltpu.sync_copy(row_hbm.at[pl.ds(base, CH)], rowv.at[p])
        pltpu.async_copy(q_hbm.at[colv.at[p]], qb, sem)
        pltpu.async_copy(k_hbm.at[rowv.at[p]], kb, sem)
        pltpu.async_copy(v_hbm.at[rowv.at[p]], vb, sem)

    def drain(j, p):
        qb, kb, vb, sem = bufs[p]
        base = wbase + j * CH
        pltpu.make_async_copy(q_hbm.at[colv.at[p]], qb, sem).wait()
        pltpu.make_async_copy(k_hbm.at[rowv.at[p]], kb, sem).wait()
        pltpu.make_async_copy(v_hbm.at[rowv.at[p]], vb, sem).wait()
        pltpu.sync_copy(qb, qg_hbm.at[pl.ds(base, CH)])
        pltpu.sync_copy(kb, kg_hbm.at[pl.ds(base, CH)])
        pltpu.sync_copy(vb, acc.at[colv.at[p]], add=True)

    assert NCHUNK % 2 == 0
    issue(0, 0)

    def step(i, carry):
        j = 2 * i
        issue(j + 1, 1)
        drain(j, 0)
        issue(j + 2, 0)
        drain(j + 1, 1)
        return carry

    lax.fori_loop(0, (NCHUNK - 2) // 2, step, 0)
    issue(NCHUNK - 1, 1)
    drain(NCHUNK - 2, 0)
    drain(NCHUNK - 1, 1)
    plsc.subcore_barrier()

    for t in range(ROWS_PER_TILE // RCH):
        r0 = base_r + t * RCH
        pltpu.sync_copy(acc.at[pl.ds(r0, RCH)], qb0)
        pltpu.sync_copy(qb0, part_hbm.at[pl.ds(c * N_PAD + r0, RCH)])


def _p2(q, k, v, row, col, zeros):
    mesh = plsc.VectorSubcoreMesh(core_axis_name="c", subcore_axis_name="s")
    fn = pl.kernel(
        _p2_body,
        out_type=(
            jax.ShapeDtypeStruct((E, D_MODEL), jnp.float32),
            jax.ShapeDtypeStruct((E, D_MODEL), jnp.float32),
            jax.ShapeDtypeStruct((NC * N_PAD, D_MODEL), jnp.float32),
        ),
        mesh=mesh,
        scratch_types=[
            pltpu.VMEM((2, CH), jnp.int32),
            pltpu.VMEM((2, CH), jnp.int32),
            pltpu.VMEM((CH, D_MODEL), jnp.float32),
            pltpu.VMEM((CH, D_MODEL), jnp.float32),
            pltpu.VMEM((CH, D_MODEL), jnp.float32),
            pltpu.VMEM((CH, D_MODEL), jnp.float32),
            pltpu.VMEM((CH, D_MODEL), jnp.float32),
            pltpu.VMEM((CH, D_MODEL), jnp.float32),
            pltpu.VMEM_SHARED((N_PAD, D_MODEL), jnp.float32),
            pltpu.SemaphoreType.DMA,
            pltpu.SemaphoreType.DMA,
        ],
    )
    return fn(q, k, v, row, col, zeros)


def _p3_body(qg, kg, er, sp, ksum, Wrel, brel, wsp, bsp, attw_o):
    rel = _dot(er[...], Wrel[...]) + brel[...]
    qe = qg[...] + rel
    ke = kg[...] + rel
    rows = lax.broadcasted_iota(jnp.int32, (D_MODEL, H), 0) // DEPTH
    cols = lax.broadcasted_iota(jnp.int32, (D_MODEL, H), 1)
    hm = (rows == cols).astype(jnp.float32)
    c = jnp.float32(1.0 / np.sqrt(np.sqrt(float(H))))
    num = _dot_f32(qe * ke, hm) * c + sp[...] * wsp[...] + bsp[...]
    qg16 = qg[...].astype(jnp.bfloat16).astype(jnp.float32)
    ks16 = ksum[...].astype(jnp.bfloat16).astype(jnp.float32)
    norm = _dot_f32(qg16 * ks16, hm)
    attw_o[...] = num / norm


def _p3(qg, kg, edge_rel, sp_value, ksum, Wrel, brel, wsp, bsp):
    grid = E // BE
    full = lambda shape: pl.BlockSpec(shape, lambda i: (0, 0))
    return pl.pallas_call(
        _p3_body,
        grid=(grid,),
        in_specs=[
            pl.BlockSpec((BE, D_MODEL), lambda i: (i, 0)),
            pl.BlockSpec((BE, D_MODEL), lambda i: (i, 0)),
            pl.BlockSpec((BE, D_EDGE), lambda i: (i, 0)),
            pl.BlockSpec((BE, 1), lambda i: (i, 0)),
            full((1, D_MODEL)),
            full((D_EDGE, D_MODEL)), full((1, D_MODEL)),
            full((1, H)), full((1, H)),
        ],
        out_specs=pl.BlockSpec((BE, H), lambda i: (i, 0)),
        out_shape=jax.ShapeDtypeStruct((E, H), jnp.float32),
    )(qg, kg, edge_rel, sp_value, ksum, Wrel, brel, wsp, bsp)


def _p4_body(part, feat, Wd, bd, g2, b2, W1, bf1, W2, bf2, out_o):
    agg = part[0] + part[1]
    attn_out = _dot(agg, Wd[...]) + bd[...]
    out1 = attn_out + feat[...]
    m = jnp.mean(out1, axis=-1, keepdims=True)
    var = jnp.mean(jnp.square(out1 - m), axis=-1, keepdims=True)
    t = (out1 - m) / jnp.sqrt(var + 1e-6) * g2[...] + b2[...]
    ffn = _dot(jnp.maximum(_dot(t, W1[...]) + bf1[...], 0.0), W2[...]) + bf2[...]
    out_o[...] = out1 + ffn


def _p4(partials, feature, Wd, bd, g2, b2, W1, bf1, W2, bf2):
    grid = N // BN
    full = lambda shape: pl.BlockSpec(shape, lambda *_: tuple(0 for _ in shape))
    return pl.pallas_call(
        _p4_body,
        grid=(grid,),
        in_specs=[
            pl.BlockSpec((2, BN, D_MODEL), lambda i: (0, i, 0)),
            pl.BlockSpec((BN, D_MODEL), lambda i: (i, 0)),
            full((D_MODEL, D_MODEL)), full((1, D_MODEL)),
            full((1, D_MODEL)), full((1, D_MODEL)),
            full((D_MODEL, D_FF)), full((1, D_FF)),
            full((D_FF, D_MODEL)), full((1, D_MODEL)),
        ],
        out_specs=pl.BlockSpec((BN, D_MODEL), lambda i: (i, 0)),
        out_shape=jax.ShapeDtypeStruct((N, D_MODEL), jnp.float32),
    )(partials, feature, Wd, bd, g2, b2, W1, bf1, W2, bf2)


def kernel(feature, sp_edge_index, sp_value, edge_rel, g1, b1, g2, b2,
           Wq, bq, Wk, bk, Wv, bv, Wd, bd, Wrel, brel, Wsp, bsp,
           W1, bf1, W2, bf2):
    r2 = lambda a: a.reshape(1, -1)
    q, k, v, ksum = _p1(feature, r2(g1), r2(b1), Wq, r2(bq), Wk, r2(bk),
                        Wv, r2(bv))
    row = sp_edge_index[0]
    col = sp_edge_index[1]
    zeros = jnp.zeros((RCH, D_MODEL), jnp.float32)
    qg, kg, partials = _p2(q, k, v, row, col, zeros)
    attw = _p3(qg, kg, edge_rel, sp_value, ksum, Wrel, r2(brel),
               Wsp.reshape(1, H), r2(bsp))
    partials = partials.reshape(NC, N_PAD, D_MODEL)[:, :N, :]
    out2 = _p4(partials, feature, Wd, r2(bd), r2(g2), r2(b2),
               W1, r2(bf1), W2, r2(bf2))
    return (out2, attw)
